# baseline (speedup 1.0000x reference)
import functools
import jax
import jax.numpy as jnp
from jax import lax
from jax.experimental import pallas as pl
from jax.experimental.pallas import tpu as pltpu
from jax.experimental.pallas import tpu_sc as plsc

B, T, D = 4, 8192, 1024
NC, NS = 2, 16
NW = NC * NS
ROWS = B * T
RPW = ROWS // NW
C = 16
NCH = RPW // C
CHUNK = C * D


def _sc_body(in_hbm, emb_hbm, out_hbm,
             in0, in1, e0, e1, sin0, sin1, sout0, sout1):
    wid = lax.axis_index("s") * NC + lax.axis_index("c")
    base_row = wid * RPW
    t0 = lax.rem(wid, 8) * RPW
    in_base = base_row * D
    emb_base = (t0 + 1) * D

    def do_chunk(g, in_v, e_v, sin, sout):
        off = g * CHUNK
        ci = pltpu.async_copy(in_hbm.at[pl.ds(in_base + off, CHUNK)], in_v, sin)
        ce = pltpu.async_copy(emb_hbm.at[pl.ds(emb_base + off, CHUNK)], e_v, sin)
        return ci, ce

    def compute(in_v, e_v):
        def body(i, _):
            s = pl.ds(i * 16, 16)
            in_v[s] = in_v[s] + e_v[s]
            return 0
        lax.fori_loop(0, CHUNK // 16, body, 0, unroll=4)

    def loop_body(k, _):
        g0 = 2 * k
        g1 = 2 * k + 1
        ci0, ce0 = do_chunk(g0, in0, e0, sin0, sout0)
        ci1, ce1 = do_chunk(g1, in1, e1, sin1, sout1)
        ci0.wait()
        ce0.wait()
        compute(in0, e0)
        co0 = pltpu.async_copy(in0, out_hbm.at[pl.ds(in_base + g0 * CHUNK, CHUNK)], sout0)
        ci1.wait()
        ce1.wait()
        compute(in1, e1)
        co1 = pltpu.async_copy(in1, out_hbm.at[pl.ds(in_base + g1 * CHUNK, CHUNK)], sout1)
        co0.wait()
        co1.wait()
        return 0

    lax.fori_loop(0, NCH // 2, loop_body, 0)


def kernel(inputs, embedding):
    Bi, Ti, Di = inputs.shape
    mesh = plsc.VectorSubcoreMesh(core_axis_name="c", subcore_axis_name="s")
    k = functools.partial(
        pl.kernel,
        mesh=mesh,
        out_type=jax.ShapeDtypeStruct((ROWS * D,), jnp.float32),
        scratch_types=[
            pltpu.VMEM((CHUNK,), jnp.float32),
            pltpu.VMEM((CHUNK,), jnp.float32),
            pltpu.VMEM((CHUNK,), jnp.float32),
            pltpu.VMEM((CHUNK,), jnp.float32),
            pltpu.SemaphoreType.DMA,
            pltpu.SemaphoreType.DMA,
            pltpu.SemaphoreType.DMA,
            pltpu.SemaphoreType.DMA,
        ],
    )(_sc_body)
    out = k(inputs.reshape(ROWS * D), embedding.reshape(-1))
    return out.reshape(Bi, Ti, Di)


# speedup vs baseline: 7.5689x; 7.5689x over previous
import jax
import jax.numpy as jnp
from jax.experimental import pallas as pl


def _add_kernel(x_ref, e1_ref, e2_ref, o_ref):
    e = jnp.concatenate([e1_ref[1:, :], e2_ref[:1, :]], axis=0)
    o_ref[...] = x_ref[...] + e[None, :, :]


def kernel(inputs, embedding):
    B, T, D = inputs.shape
    BT = 2048
    grid = (T // BT, B)
    return pl.pallas_call(
        _add_kernel,
        grid=grid,
        in_specs=[
            pl.BlockSpec((1, BT, D), lambda t, b: (b, t, 0)),
            pl.BlockSpec((BT, D), lambda t, b: (t, 0)),
            pl.BlockSpec((8, D), lambda t, b: ((t + 1) * (BT // 8), 0)),
        ],
        out_specs=pl.BlockSpec((1, BT, D), lambda t, b: (b, t, 0)),
        out_shape=jax.ShapeDtypeStruct((B, T, D), inputs.dtype),
    )(inputs, embedding, embedding)
